# baseline (speedup 1.0000x reference)
import functools

import jax
import jax.numpy as jnp
from jax import lax
from jax.experimental import pallas as pl
from jax.experimental.pallas import tpu as pltpu
from jax.experimental.pallas import tpu_sc as plsc

D_MODEL = 768
NUM_EXPERTS = 64
EXPERT_D = 3072
T = 4096
TB = 64
NCH = 128
PADDED = NCH * TB
CHUNK_LOG2 = 6


def _router_body(xf_ref, rw_ref, rb_ref, dest_ref, meta_ref, aux_ref):
    E = NUM_EXPERTS
    xf = xf_ref[...]
    rw = rw_ref[...]
    logits = lax.dot_general(xf, rw, (((1,), (1,)), ((), ())),
                             preferred_element_type=jnp.float32)
    logits = logits + rb_ref[...]

    mx = jnp.max(logits, axis=1, keepdims=True)
    col = lax.broadcasted_iota(jnp.int32, (T, E), 1)
    idx = jnp.min(jnp.where(logits == mx, col, E), axis=1, keepdims=True)

    p = jnp.exp(logits - mx)
    probs = p / jnp.sum(p, axis=1, keepdims=True)
    colsum = jnp.sum(probs, axis=0, keepdims=True)
    aux = jnp.sum((colsum * (1.0 / T) - (1.0 / E)) ** 2)
    aux_ref[...] = jnp.zeros((1, 1), jnp.float32) + aux

    oh = (col == idx).astype(jnp.float32)
    counts = jnp.sum(oh, axis=0, keepdims=True)
    nch = (counts.astype(jnp.int32) + (TB - 1)) >> CHUNK_LOG2
    nch_f = nch.astype(jnp.float32)

    r_i = lax.broadcasted_iota(jnp.int32, (E, E), 0)
    c_i = lax.broadcasted_iota(jnp.int32, (E, E), 1)
    m_excl = (r_i < c_i).astype(jnp.float32)
    m_incl = (r_i <= c_i).astype(jnp.float32)
    cs_excl = lax.dot_general(nch_f, m_excl, (((1,), (0,)), ((), ())),
                              preferred_element_type=jnp.float32)
    cs_incl = lax.dot_general(nch_f, m_incl, (((1,), (0,)), ((), ())),
                              preferred_element_type=jnp.float32)
    po = cs_excl * float(TB)

    g_i = lax.broadcasted_iota(jnp.int32, (NCH, E), 0)
    eoc = jnp.sum((g_i >= cs_incl.astype(jnp.int32)).astype(jnp.float32),
                  axis=1, keepdims=True)
    eoc = jnp.minimum(eoc, float(E - 1))
    used = jnp.sum(nch_f)
    meta = jnp.concatenate(
        [eoc, jnp.zeros((NCH, 1), jnp.float32) + used], axis=0)
    meta_ref[...] = meta.astype(jnp.int32)

    CB = 128
    rr = lax.broadcasted_iota(jnp.int32, (CB, CB), 0)
    cc = lax.broadcasted_iota(jnp.int32, (CB, CB), 1)
    tri = (rr >= cc).astype(jnp.float32)
    carry = jnp.zeros((1, E), jnp.float32)
    for c in range(T // CB):
        ohb = oh[c * CB:(c + 1) * CB, :]
        csum = lax.dot_general(tri, ohb, (((1,), (0,)), ((), ())),
                               preferred_element_type=jnp.float32) + carry
        carry = carry + jnp.sum(ohb, axis=0, keepdims=True)
        destb = jnp.sum(ohb * (po + csum - 1.0), axis=1, keepdims=True)
        dest_ref[c * CB:(c + 1) * CB, :] = destb.astype(jnp.int32)


_router_call = pl.pallas_call(
    _router_body,
    out_shape=(
        jax.ShapeDtypeStruct((T, 1), jnp.int32),
        jax.ShapeDtypeStruct((2 * NCH, 1), jnp.int32),
        jax.ShapeDtypeStruct((1, 1), jnp.float32),
    ),
)


_NC, _NS = 2, 16
_NW = _NC * _NS
_TPW = T // _NW


def _dispatch_body(xf_hbm, dest_hbm, xs_hbm, idx_v, rows_v, sem):
    wid = lax.axis_index("s") * _NC + lax.axis_index("c")
    base = wid * _TPW
    pltpu.sync_copy(dest_hbm.at[pl.ds(base, _TPW)], idx_v)
    pltpu.sync_copy(xf_hbm.at[pl.ds(base, _TPW)], rows_v)
    pltpu.async_copy(rows_v, xs_hbm.at[idx_v], sem).wait()


def _combine_body(ys_hbm, dest_hbm, out_hbm, idx_v, rows_v, sem):
    wid = lax.axis_index("s") * _NC + lax.axis_index("c")
    base = wid * _TPW
    pltpu.sync_copy(dest_hbm.at[pl.ds(base, _TPW)], idx_v)
    pltpu.async_copy(ys_hbm.at[idx_v], rows_v, sem).wait()
    pltpu.sync_copy(rows_v, out_hbm.at[pl.ds(base, _TPW)])


@functools.cache
def _sc_calls():
    mesh = plsc.VectorSubcoreMesh(core_axis_name="c", subcore_axis_name="s")
    scratch = [
        pltpu.VMEM((_TPW,), jnp.int32),
        pltpu.VMEM((_TPW, D_MODEL), jnp.float32),
        pltpu.SemaphoreType.DMA,
    ]
    dispatch = pl.kernel(
        _dispatch_body,
        out_type=jax.ShapeDtypeStruct((PADDED, D_MODEL), jnp.float32),
        mesh=mesh,
        scratch_types=scratch,
    )
    combine = pl.kernel(
        _combine_body,
        out_type=jax.ShapeDtypeStruct((T, D_MODEL), jnp.float32),
        mesh=mesh,
        scratch_types=scratch,
    )
    return dispatch, combine


def _ffn_body(meta_ref, xs_ref, w1_ref, b1_ref, w2_ref, b2_ref, ys_ref):
    g = pl.program_id(0)

    @pl.when(g < meta_ref[NCH])
    def _():
        xb = xs_ref[...].astype(jnp.bfloat16)
        h = lax.dot_general(xb, w1_ref[0].astype(jnp.bfloat16),
                            (((1,), (1,)), ((), ())),
                            preferred_element_type=jnp.float32)
        h = jnp.maximum(h + b1_ref[0], 0.0).astype(jnp.bfloat16)
        y = lax.dot_general(h, w2_ref[0].astype(jnp.bfloat16),
                            (((1,), (1,)), ((), ())),
                            preferred_element_type=jnp.float32)
        ys_ref[...] = y + b2_ref[0]


_ffn_call = pl.pallas_call(
    _ffn_body,
    grid_spec=pltpu.PrefetchScalarGridSpec(
        num_scalar_prefetch=1,
        grid=(NCH,),
        in_specs=[
            pl.BlockSpec((TB, D_MODEL), lambda g, m: (g, 0)),
            pl.BlockSpec((1, EXPERT_D, D_MODEL), lambda g, m: (m[g], 0, 0)),
            pl.BlockSpec((1, 1, EXPERT_D), lambda g, m: (m[g], 0, 0)),
            pl.BlockSpec((1, D_MODEL, EXPERT_D), lambda g, m: (m[g], 0, 0)),
            pl.BlockSpec((1, 1, D_MODEL), lambda g, m: (m[g], 0, 0)),
        ],
        out_specs=pl.BlockSpec((TB, D_MODEL), lambda g, m: (g, 0)),
    ),
    out_shape=jax.ShapeDtypeStruct((PADDED, D_MODEL), jnp.float32),
)


def kernel(x, router_w, router_b, W1, b1, W2, b2):
    b, s, d = x.shape
    xf = x.reshape(-1, d)
    dest2d, meta2d, aux2d = _router_call(xf, router_w, router_b.reshape(1, -1))
    dest = dest2d.reshape(-1)
    meta = meta2d.reshape(-1)
    dispatch_call, combine_call = _sc_calls()
    xs = dispatch_call(xf, dest)
    ys = _ffn_call(meta, xs, W1, b1.reshape(NUM_EXPERTS, 1, EXPERT_D),
                   W2, b2.reshape(NUM_EXPERTS, 1, D_MODEL))
    out = combine_call(ys, dest)
    return out.reshape(b, s, d), aux2d.reshape(())


# speedup vs baseline: 1.2745x; 1.2745x over previous
import functools

import jax
import jax.numpy as jnp
from jax import lax
from jax.experimental import pallas as pl
from jax.experimental.pallas import tpu as pltpu
from jax.experimental.pallas import tpu_sc as plsc

D_MODEL = 768
NUM_EXPERTS = 64
EXPERT_D = 3072
T = 4096
TB = 128
NCH = 96
PADDED = NCH * TB
CHUNK_LOG2 = 7


def _router_body(xf_ref, rw_ref, rb_ref, dest_ref, meta_ref, aux_ref):
    E = NUM_EXPERTS
    xf = xf_ref[...]
    rw = rw_ref[...]
    logits = lax.dot_general(xf, rw, (((1,), (1,)), ((), ())),
                             preferred_element_type=jnp.float32)
    logits = logits + rb_ref[...]

    mx = jnp.max(logits, axis=1, keepdims=True)
    col = lax.broadcasted_iota(jnp.int32, (T, E), 1)
    idx = jnp.min(jnp.where(logits == mx, col, E), axis=1, keepdims=True)

    p = jnp.exp(logits - mx)
    probs = p / jnp.sum(p, axis=1, keepdims=True)
    colsum = jnp.sum(probs, axis=0, keepdims=True)
    aux = jnp.sum((colsum * (1.0 / T) - (1.0 / E)) ** 2)
    aux_ref[...] = jnp.zeros((1, 1), jnp.float32) + aux

    oh = (col == idx).astype(jnp.float32)
    counts = jnp.sum(oh, axis=0, keepdims=True)
    nch = (counts.astype(jnp.int32) + (TB - 1)) >> CHUNK_LOG2
    nch_f = nch.astype(jnp.float32)

    r_i = lax.broadcasted_iota(jnp.int32, (E, E), 0)
    c_i = lax.broadcasted_iota(jnp.int32, (E, E), 1)
    m_excl = (r_i < c_i).astype(jnp.float32)
    m_incl = (r_i <= c_i).astype(jnp.float32)
    cs_excl = lax.dot_general(nch_f, m_excl, (((1,), (0,)), ((), ())),
                              preferred_element_type=jnp.float32)
    cs_incl = lax.dot_general(nch_f, m_incl, (((1,), (0,)), ((), ())),
                              preferred_element_type=jnp.float32)
    po = cs_excl * float(TB)

    g_i = lax.broadcasted_iota(jnp.int32, (NCH, E), 0)
    eoc = jnp.sum((g_i >= cs_incl.astype(jnp.int32)).astype(jnp.float32),
                  axis=1, keepdims=True)
    eoc = jnp.minimum(eoc, float(E - 1))
    used = jnp.sum(nch_f)
    meta = jnp.concatenate(
        [eoc, jnp.zeros((NCH, 1), jnp.float32) + used], axis=0)
    meta_ref[...] = meta.astype(jnp.int32)

    CB = 128
    rr = lax.broadcasted_iota(jnp.int32, (CB, CB), 0)
    cc = lax.broadcasted_iota(jnp.int32, (CB, CB), 1)
    tri = (rr >= cc).astype(jnp.float32)
    carry = jnp.zeros((1, E), jnp.float32)
    for c in range(T // CB):
        ohb = oh[c * CB:(c + 1) * CB, :]
        csum = lax.dot_general(tri, ohb, (((1,), (0,)), ((), ())),
                               preferred_element_type=jnp.float32) + carry
        carry = carry + jnp.sum(ohb, axis=0, keepdims=True)
        destb = jnp.sum(ohb * (po + csum - 1.0), axis=1, keepdims=True)
        dest_ref[c * CB:(c + 1) * CB, :] = destb.astype(jnp.int32)


_router_call = pl.pallas_call(
    _router_body,
    out_shape=(
        jax.ShapeDtypeStruct((T, 1), jnp.int32),
        jax.ShapeDtypeStruct((2 * NCH, 1), jnp.int32),
        jax.ShapeDtypeStruct((1, 1), jnp.float32),
    ),
)


_NC, _NS = 2, 16
_NW = _NC * _NS
_TPW = T // _NW


def _dispatch_body(xf_hbm, dest_hbm, xs_hbm, idx_v, rows_v, sem):
    wid = lax.axis_index("s") * _NC + lax.axis_index("c")
    base = wid * _TPW
    pltpu.sync_copy(dest_hbm.at[pl.ds(base, _TPW)], idx_v)
    pltpu.sync_copy(xf_hbm.at[pl.ds(base, _TPW)], rows_v)
    pltpu.async_copy(rows_v, xs_hbm.at[idx_v], sem).wait()


def _combine_body(ys_hbm, dest_hbm, out_hbm, idx_v, rows_v, sem):
    wid = lax.axis_index("s") * _NC + lax.axis_index("c")
    base = wid * _TPW
    pltpu.sync_copy(dest_hbm.at[pl.ds(base, _TPW)], idx_v)
    pltpu.async_copy(ys_hbm.at[idx_v], rows_v, sem).wait()
    pltpu.sync_copy(rows_v, out_hbm.at[pl.ds(base, _TPW)])


@functools.cache
def _sc_calls():
    mesh = plsc.VectorSubcoreMesh(core_axis_name="c", subcore_axis_name="s")
    scratch = [
        pltpu.VMEM((_TPW,), jnp.int32),
        pltpu.VMEM((_TPW, D_MODEL), jnp.float32),
        pltpu.SemaphoreType.DMA,
    ]
    dispatch = pl.kernel(
        _dispatch_body,
        out_type=jax.ShapeDtypeStruct((PADDED, D_MODEL), jnp.float32),
        mesh=mesh,
        scratch_types=scratch,
    )
    combine = pl.kernel(
        _combine_body,
        out_type=jax.ShapeDtypeStruct((T, D_MODEL), jnp.float32),
        mesh=mesh,
        scratch_types=scratch,
    )
    return dispatch, combine


def _ffn_body(meta_ref, xs_ref, w1_ref, b1_ref, w2_ref, b2_ref, ys_ref):
    g = pl.program_id(0)

    @pl.when(g < meta_ref[NCH])
    def _():
        xb = xs_ref[...]
        h = lax.dot_general(xb, w1_ref[0], (((1,), (1,)), ((), ())),
                            preferred_element_type=jnp.float32)
        h = jnp.maximum(h + b1_ref[0], 0.0)
        y = lax.dot_general(h, w2_ref[0], (((1,), (1,)), ((), ())),
                            preferred_element_type=jnp.float32)
        ys_ref[...] = y + b2_ref[0]


_ffn_call = pl.pallas_call(
    _ffn_body,
    grid_spec=pltpu.PrefetchScalarGridSpec(
        num_scalar_prefetch=1,
        grid=(NCH,),
        in_specs=[
            pl.BlockSpec((TB, D_MODEL), lambda g, m: (g, 0)),
            pl.BlockSpec((1, EXPERT_D, D_MODEL), lambda g, m: (m[g], 0, 0)),
            pl.BlockSpec((1, 1, EXPERT_D), lambda g, m: (m[g], 0, 0)),
            pl.BlockSpec((1, D_MODEL, EXPERT_D), lambda g, m: (m[g], 0, 0)),
            pl.BlockSpec((1, 1, D_MODEL), lambda g, m: (m[g], 0, 0)),
        ],
        out_specs=pl.BlockSpec((TB, D_MODEL), lambda g, m: (g, 0)),
    ),
    out_shape=jax.ShapeDtypeStruct((PADDED, D_MODEL), jnp.float32),
)


def kernel(x, router_w, router_b, W1, b1, W2, b2):
    b, s, d = x.shape
    xf = x.reshape(-1, d)
    dest2d, meta2d, aux2d = _router_call(xf, router_w, router_b.reshape(1, -1))
    dest = dest2d.reshape(-1)
    meta = meta2d.reshape(-1)
    dispatch_call, combine_call = _sc_calls()
    xs = dispatch_call(xf, dest)
    ys = _ffn_call(meta, xs, W1, b1.reshape(NUM_EXPERTS, 1, EXPERT_D),
                   W2, b2.reshape(NUM_EXPERTS, 1, D_MODEL))
    out = combine_call(ys, dest)
    return out.reshape(b, s, d), aux2d.reshape(())
